# baseline (speedup 1.0000x reference)
import jax
import jax.numpy as jnp
from jax.experimental import pallas as pl
from jax.experimental.pallas import tpu as pltpu

_B, _D, _T = 32, 64, 576
_K = 1024
_COMMITMENT_COST = 0.25
_VQ_COST = 1.0
_STEPS = _B // 2 + 1


def _min_onehot(score):
    minv = jnp.min(score, axis=0)
    oh = jnp.where(score == minv[None, :], 1.0, 0.0)
    return minv, oh.astype(jnp.bfloat16)


def _vq_kernel(x_ref, e_ref, q_ref, idx_ref, sse_ref,
               enorm_ref, eaug_ref, g0_ref, g1_ref, oh_ref):
    s = pl.program_id(0)
    emb = e_ref[...]

    @pl.when(s == 0)
    def _first():
        enorm_ref[...] = jnp.sum(emb * emb, axis=1, keepdims=True)
        k = jax.lax.broadcasted_iota(jnp.int32, (_K, 1), 0)
        khi = ((k >> 5) << 5).astype(jnp.float32)
        klo = (k & 31).astype(jnp.float32)
        eaug_ref[...] = jnp.concatenate(
            [emb, khi, klo], axis=1).astype(jnp.bfloat16)
        sse_ref[...] = jnp.zeros((1, 1), jnp.float32)

    enorm = enorm_ref[...]
    eaug = eaug_ref[...]
    dot_kk = (((0,), (0,)), ((), ()))
    dot_kd = (((1,), (0,)), ((), ()))

    minv1, oh1 = _min_onehot(enorm - 2.0 * g1_ref[...])
    qa_e = jax.lax.dot_general(eaug, oh_ref[...], dot_kk,
                               preferred_element_type=jnp.float32)
    q_ref[0] = qa_e[:_D]
    idx_ref[0, 0] = (qa_e[_D] + qa_e[_D + 1]).astype(jnp.int32)
    g0_ref[...] = jax.lax.dot_general(emb, x_ref[0], dot_kd,
                                      preferred_element_type=jnp.float32)
    g1_ref[...] = jax.lax.dot_general(emb, x_ref[1], dot_kd,
                                      preferred_element_type=jnp.float32)
    minv0, oh0 = _min_onehot(enorm - 2.0 * g0_ref[...])
    oh_ref[...] = oh0
    qa_o = jax.lax.dot_general(eaug, oh1, dot_kk,
                               preferred_element_type=jnp.float32)
    q_ref[1] = qa_o[:_D]
    idx_ref[1, 0] = (qa_o[_D] + qa_o[_D + 1]).astype(jnp.int32)

    xb = x_ref[...]
    sse_ref[...] += (
        jnp.where(s < _STEPS - 1, jnp.sum(xb * xb) + jnp.sum(minv0), 0.0)
        + jnp.where(s >= 1, jnp.sum(minv1), 0.0)
    ).reshape(1, 1)


def kernel(x, embeddings):
    q, idx, sse = pl.pallas_call(
        _vq_kernel,
        grid=(_STEPS,),
        in_specs=[
            pl.BlockSpec((2, _D, _T), lambda s: (jnp.minimum(s, _STEPS - 2), 0, 0)),
            pl.BlockSpec((_K, _D), lambda s: (0, 0)),
        ],
        out_specs=[
            pl.BlockSpec((2, _D, _T), lambda s: (jnp.maximum(s - 1, 0), 0, 0)),
            pl.BlockSpec((2, 1, _T), lambda s: (jnp.maximum(s - 1, 0), 0, 0)),
            pl.BlockSpec((1, 1), lambda s: (0, 0)),
        ],
        out_shape=[
            jax.ShapeDtypeStruct((_B, _D, _T), jnp.float32),
            jax.ShapeDtypeStruct((_B, 1, _T), jnp.int32),
            jax.ShapeDtypeStruct((1, 1), jnp.float32),
        ],
        scratch_shapes=[
            pltpu.VMEM((_K, 1), jnp.float32),
            pltpu.VMEM((_K, _D + 2), jnp.bfloat16),
            pltpu.VMEM((_K, _T), jnp.float32),
            pltpu.VMEM((_K, _T), jnp.float32),
            pltpu.VMEM((_K, _T), jnp.bfloat16),
        ],
    )(x, embeddings)
    e = sse[0, 0] / (_B * _T * _D)
    loss_commit = _COMMITMENT_COST * e
    loss_vq = _VQ_COST * e
    return q, loss_commit, loss_vq, idx.reshape(_B * _T)


# speedup vs baseline: 1.0308x; 1.0308x over previous
import jax
import jax.numpy as jnp
from jax.experimental import pallas as pl
from jax.experimental.pallas import tpu as pltpu

_B, _D, _T = 32, 64, 576
_K = 1024
_COMMITMENT_COST = 0.25
_VQ_COST = 1.0
_STEPS = _B // 2 + 1


def _argmin_onehot(score):
    idx = jnp.argmin(score, axis=0).astype(jnp.int32)
    iota_k = jax.lax.broadcasted_iota(jnp.int32, (_K, _T), 0)
    return idx, jnp.where(iota_k == idx[None, :], 1.0, 0.0)


def _vq_kernel(x_ref, e_ref, q_ref, idx_ref, sse_ref,
               enorm_ref, g0_ref, g1_ref, oh_ref, idxs_ref, xs_ref):
    s = pl.program_id(0)
    emb = e_ref[...]

    @pl.when(s == 0)
    def _first():
        enorm_ref[...] = jnp.sum(emb * emb, axis=1, keepdims=True)
        sse_ref[...] = jnp.zeros((1, 1), jnp.float32)

    enorm = enorm_ref[...]
    dot_kk = (((0,), (0,)), ((), ()))
    dot_kd = (((1,), (0,)), ((), ()))

    idx1, oh1 = _argmin_onehot(enorm - 2.0 * g1_ref[...])
    qa_e = jax.lax.dot_general(emb, oh_ref[...], dot_kk,
                               preferred_element_type=jnp.float32)
    q_ref[0] = qa_e
    idx_ref[0, 0] = idxs_ref[0, 0]
    d_e = xs_ref[0] - qa_e
    qa_o = jax.lax.dot_general(emb, oh1, dot_kk,
                               preferred_element_type=jnp.float32)
    q_ref[1] = qa_o
    idx_ref[1, 0] = idx1
    d_o = xs_ref[1] - qa_o
    sse_ref[...] += jnp.where(
        s >= 1, jnp.sum(d_e * d_e) + jnp.sum(d_o * d_o), 0.0).reshape(1, 1)

    g0_ref[...] = jax.lax.dot_general(emb, x_ref[0], dot_kd,
                                      preferred_element_type=jnp.float32)
    g1_ref[...] = jax.lax.dot_general(emb, x_ref[1], dot_kd,
                                      preferred_element_type=jnp.float32)
    xs_ref[...] = x_ref[...]

    idx0, oh0 = _argmin_onehot(enorm - 2.0 * g0_ref[...])
    oh_ref[...] = oh0
    idxs_ref[0, 0] = idx0


def kernel(x, embeddings):
    q, idx, sse = pl.pallas_call(
        _vq_kernel,
        grid=(_STEPS,),
        in_specs=[
            pl.BlockSpec((2, _D, _T), lambda s: (jnp.minimum(s, _STEPS - 2), 0, 0)),
            pl.BlockSpec((_K, _D), lambda s: (0, 0)),
        ],
        out_specs=[
            pl.BlockSpec((2, _D, _T), lambda s: (jnp.maximum(s - 1, 0), 0, 0)),
            pl.BlockSpec((2, 1, _T), lambda s: (jnp.maximum(s - 1, 0), 0, 0)),
            pl.BlockSpec((1, 1), lambda s: (0, 0)),
        ],
        out_shape=[
            jax.ShapeDtypeStruct((_B, _D, _T), jnp.float32),
            jax.ShapeDtypeStruct((_B, 1, _T), jnp.int32),
            jax.ShapeDtypeStruct((1, 1), jnp.float32),
        ],
        scratch_shapes=[
            pltpu.VMEM((_K, 1), jnp.float32),
            pltpu.VMEM((_K, _T), jnp.float32),
            pltpu.VMEM((_K, _T), jnp.float32),
            pltpu.VMEM((_K, _T), jnp.float32),
            pltpu.VMEM((1, 1, _T), jnp.int32),
            pltpu.VMEM((2, _D, _T), jnp.float32),
        ],
    )(x, embeddings)
    e = sse[0, 0] / (_B * _T * _D)
    loss_commit = _COMMITMENT_COST * e
    loss_vq = _VQ_COST * e
    return q, loss_commit, loss_vq, idx.reshape(_B * _T)


# speedup vs baseline: 1.0681x; 1.0362x over previous
import jax
import jax.numpy as jnp
from jax.experimental import pallas as pl
from jax.experimental.pallas import tpu as pltpu

_B, _D, _T = 32, 64, 576
_K = 1024
_DA = _D + 3
_COMMITMENT_COST = 0.25
_VQ_COST = 1.0
_STEPS = _B // 2 + 1


def _argmin_onehot(score):
    idx = jnp.argmin(score, axis=0).astype(jnp.int32)
    iota_k = jax.lax.broadcasted_iota(jnp.int32, (_K, _T), 0)
    return idx, jnp.where(iota_k == idx[None, :], 1.0, 0.0)


def _vq_kernel(x_ref, e_ref, q_ref, idx_ref, sse_ref,
               ea_ref, xa_ref, s0_ref, s1_ref, oh_ref, idxs_ref, xs_ref):
    s = pl.program_id(0)
    emb = e_ref[...]

    @pl.when(s == 0)
    def _first():
        enorm = jnp.sum(emb * emb, axis=1, keepdims=True)
        e1 = enorm.astype(jnp.bfloat16).astype(jnp.float32)
        r = enorm - e1
        e2 = r.astype(jnp.bfloat16).astype(jnp.float32)
        e3 = (r - e2).astype(jnp.bfloat16).astype(jnp.float32)
        ea_ref[...] = jnp.concatenate([-2.0 * emb, e1, e2, e3], axis=1)
        xa_ref[:, _D:, :] = jnp.ones((2, 3, _T), jnp.float32)
        sse_ref[...] = jnp.zeros((1, 1), jnp.float32)

    ea = ea_ref[...]
    dot_kk = (((0,), (0,)), ((), ()))
    dot_ka = (((1,), (0,)), ((), ()))

    idx1, oh1 = _argmin_onehot(s1_ref[...])
    qa_e = jax.lax.dot_general(emb, oh_ref[...], dot_kk,
                               preferred_element_type=jnp.float32)
    q_ref[0] = qa_e
    idx_ref[0, 0] = idxs_ref[0, 0]
    d_e = xs_ref[0] - qa_e
    qa_o = jax.lax.dot_general(emb, oh1, dot_kk,
                               preferred_element_type=jnp.float32)
    q_ref[1] = qa_o
    idx_ref[1, 0] = idx1
    d_o = xs_ref[1] - qa_o
    sse_ref[...] += jnp.where(
        s >= 1, jnp.sum(d_e * d_e) + jnp.sum(d_o * d_o), 0.0).reshape(1, 1)

    xa_ref[0, :_D, :] = x_ref[0]
    xa_ref[1, :_D, :] = x_ref[1]
    s0_ref[...] = jax.lax.dot_general(ea, xa_ref[0], dot_ka,
                                      preferred_element_type=jnp.float32)
    s1_ref[...] = jax.lax.dot_general(ea, xa_ref[1], dot_ka,
                                      preferred_element_type=jnp.float32)
    xs_ref[...] = x_ref[...]

    idx0, oh0 = _argmin_onehot(s0_ref[...])
    oh_ref[...] = oh0
    idxs_ref[0, 0] = idx0


def kernel(x, embeddings):
    q, idx, sse = pl.pallas_call(
        _vq_kernel,
        grid=(_STEPS,),
        in_specs=[
            pl.BlockSpec((2, _D, _T), lambda s: (jnp.minimum(s, _STEPS - 2), 0, 0)),
            pl.BlockSpec((_K, _D), lambda s: (0, 0)),
        ],
        out_specs=[
            pl.BlockSpec((2, _D, _T), lambda s: (jnp.maximum(s - 1, 0), 0, 0)),
            pl.BlockSpec((2, 1, _T), lambda s: (jnp.maximum(s - 1, 0), 0, 0)),
            pl.BlockSpec((1, 1), lambda s: (0, 0)),
        ],
        out_shape=[
            jax.ShapeDtypeStruct((_B, _D, _T), jnp.float32),
            jax.ShapeDtypeStruct((_B, 1, _T), jnp.int32),
            jax.ShapeDtypeStruct((1, 1), jnp.float32),
        ],
        scratch_shapes=[
            pltpu.VMEM((_K, _DA), jnp.float32),
            pltpu.VMEM((2, _DA, _T), jnp.float32),
            pltpu.VMEM((_K, _T), jnp.float32),
            pltpu.VMEM((_K, _T), jnp.float32),
            pltpu.VMEM((_K, _T), jnp.float32),
            pltpu.VMEM((1, 1, _T), jnp.int32),
            pltpu.VMEM((2, _D, _T), jnp.float32),
        ],
    )(x, embeddings)
    e = sse[0, 0] / (_B * _T * _D)
    loss_commit = _COMMITMENT_COST * e
    loss_vq = _VQ_COST * e
    return q, loss_commit, loss_vq, idx.reshape(_B * _T)
